# baseline (speedup 1.0000x reference)
import functools

import jax
import jax.numpy as jnp
from jax import lax
from jax.experimental import pallas as pl
from jax.experimental.pallas import tpu as pltpu
from jax.experimental.pallas import tpu_sc as plsc

_DIM = 1024
_SEQ = 4096
_SCALE = _DIM ** (-0.5)
_NC, _NS, _L = 2, 16, 16
_NW = _NC * _NS
_ROWS_W = _SEQ // _NW
_CROWS = 16
_NCHUNK = _ROWS_W // _CROWS
_NBUF = 6
_UNROLL = 16
_NVEC = _DIM // (_L * _UNROLL)

_mesh = plsc.VectorSubcoreMesh(core_axis_name="c", subcore_axis_name="s")


@functools.partial(
    pl.kernel,
    mesh=_mesh,
    out_type=jax.ShapeDtypeStruct((_SEQ, _DIM), jnp.float32),
    scratch_types=(
        [pltpu.VMEM((_CROWS, _DIM), jnp.float32) for _ in range(_NBUF)]
        + [pltpu.SemaphoreType.DMA for _ in range(2 * _NBUF)]
    ),
)
def _sc_scale_copy(emb_hbm, out_hbm, *scratch):
    bufs = scratch[:_NBUF]
    isems = scratch[_NBUF:2 * _NBUF]
    osems = scratch[2 * _NBUF:]
    wid = lax.axis_index("s") * _NC + lax.axis_index("c")
    base = wid * _ROWS_W

    def in_copy(c):
        src = emb_hbm.at[pl.ds(base + c * _CROWS, _CROWS)]
        return pltpu.make_async_copy(src, bufs[c % _NBUF], isems[c % _NBUF])

    def out_copy(c):
        dst = out_hbm.at[pl.ds(base + c * _CROWS, _CROWS)]
        return pltpu.make_async_copy(bufs[c % _NBUF], dst, osems[c % _NBUF])

    def compute(b):
        buf = bufs[b]

        def row(r, outer):
            brow = buf.at[r]

            def vec(j, inner):
                off = j * (_L * _UNROLL)
                for u in range(_UNROLL):
                    sl = pl.ds(off + u * _L, _L)
                    brow[sl] = brow[sl] * _SCALE
                return inner

            lax.fori_loop(0, _NVEC, vec, 0)
            return outer

        lax.fori_loop(0, _CROWS, row, 0)

    for c in range(min(_NBUF, _NCHUNK)):
        in_copy(c).start()

    for c in range(_NCHUNK):
        if c >= _NBUF:
            out_copy(c - _NBUF).wait()
            in_copy(c).start()
        in_copy(c).wait()
        compute(c % _NBUF)
        out_copy(c).start()

    for c in range(max(0, _NCHUNK - _NBUF), _NCHUNK):
        out_copy(c).wait()


def kernel(x, emb):
    del x
    return _sc_scale_copy(emb)


# speedup vs baseline: 1.2027x; 1.2027x over previous
import functools

import jax
import jax.numpy as jnp
from jax import lax
from jax.experimental import pallas as pl
from jax.experimental.pallas import tpu as pltpu
from jax.experimental.pallas import tpu_sc as plsc

_DIM = 1024
_SEQ = 4096
_SCALE = _DIM ** (-0.5)
_NC, _NS, _L = 2, 16, 16
_NW = _NC * _NS
_ROWS_W = _SEQ // _NW
_CROWS = 16
_NCHUNK = _ROWS_W // _CROWS
_NBUF = 3
_UNROLL = 16
_NVEC = _DIM // (_L * _UNROLL)

_mesh = plsc.VectorSubcoreMesh(core_axis_name="c", subcore_axis_name="s")


@functools.partial(
    pl.kernel,
    mesh=_mesh,
    out_type=jax.ShapeDtypeStruct((_SEQ, _DIM), jnp.float32),
    scratch_types=(
        [pltpu.VMEM((_CROWS, _DIM), jnp.float32) for _ in range(2 * _NBUF)]
        + [pltpu.SemaphoreType.DMA for _ in range(2 * _NBUF)]
    ),
)
def _sc_scale_copy(emb_hbm, out_hbm, *scratch):
    wid = lax.axis_index("s") * _NC + lax.axis_index("c")
    base = wid * _ROWS_W
    ibufs = scratch[:_NBUF]
    obufs = scratch[_NBUF:2 * _NBUF]
    isems = scratch[2 * _NBUF:3 * _NBUF]
    osems = scratch[3 * _NBUF:]

    def in_copy(c, p):
        src = emb_hbm.at[pl.ds(base + c * _CROWS, _CROWS)]
        return pltpu.make_async_copy(src, ibufs[p], isems[p])

    def out_copy(c, p):
        dst = out_hbm.at[pl.ds(base + c * _CROWS, _CROWS)]
        return pltpu.make_async_copy(obufs[p], dst, osems[p])

    def compute(p):
        src, dst = ibufs[p], obufs[p]

        def row(r, outer):
            srow, drow = src.at[r], dst.at[r]

            def vec(j, inner):
                b = j * (_L * _UNROLL)
                for u in range(_UNROLL):
                    sl = pl.ds(b + u * _L, _L)
                    drow[sl] = srow[sl] * _SCALE
                return inner

            lax.fori_loop(0, _NVEC, vec, 0)
            return outer

        lax.fori_loop(0, _CROWS, row, 0)

    _DEPTH = len(ibufs)
    for c in range(_DEPTH):
        in_copy(c, c % _DEPTH).start()

    for c in range(_NCHUNK):
        p = c % _DEPTH
        if c >= _DEPTH:
            out_copy(c - _DEPTH, p).wait()
        in_copy(c, p).wait()
        compute(p)
        out_copy(c, p).start()
        if c + _DEPTH < _NCHUNK:
            in_copy(c + _DEPTH, p).start()

    for c in range(_NCHUNK - _DEPTH, _NCHUNK):
        out_copy(c, c % _DEPTH).wait()


def kernel(x, emb):
    del x
    return _sc_scale_copy(emb)


# speedup vs baseline: 1.2773x; 1.0621x over previous
import functools

import jax
import jax.numpy as jnp
from jax import lax
from jax.experimental import pallas as pl
from jax.experimental.pallas import tpu as pltpu
from jax.experimental.pallas import tpu_sc as plsc

_DIM = 1024
_SEQ = 4096
_SCALE = _DIM ** (-0.5)
_NC, _NS, _L = 2, 16, 16
_NW = _NC * _NS
_ROWS_W = _SEQ // _NW
_CROWS = 8
_NCHUNK = _ROWS_W // _CROWS
_NBUF = 4
_UNROLL = 16
_NVEC = _DIM // (_L * _UNROLL)

_mesh = plsc.VectorSubcoreMesh(core_axis_name="c", subcore_axis_name="s")


@functools.partial(
    pl.kernel,
    mesh=_mesh,
    out_type=jax.ShapeDtypeStruct((_SEQ, _DIM), jnp.float32),
    scratch_types=(
        [pltpu.VMEM((_CROWS, _DIM), jnp.float32) for _ in range(2 * _NBUF)]
        + [pltpu.SemaphoreType.DMA for _ in range(2 * _NBUF)]
    ),
)
def _sc_scale_copy(emb_hbm, out_hbm, *scratch):
    wid = lax.axis_index("s") * _NC + lax.axis_index("c")
    base = wid * _ROWS_W
    ibufs = scratch[:_NBUF]
    obufs = scratch[_NBUF:2 * _NBUF]
    isems = scratch[2 * _NBUF:3 * _NBUF]
    osems = scratch[3 * _NBUF:]

    def in_copy(c, p):
        src = emb_hbm.at[pl.ds(base + c * _CROWS, _CROWS)]
        return pltpu.make_async_copy(src, ibufs[p], isems[p])

    def out_copy(c, p):
        dst = out_hbm.at[pl.ds(base + c * _CROWS, _CROWS)]
        return pltpu.make_async_copy(obufs[p], dst, osems[p])

    def compute(p):
        src, dst = ibufs[p], obufs[p]

        def row(r, outer):
            srow, drow = src.at[r], dst.at[r]

            def vec(j, inner):
                b = j * (_L * _UNROLL)
                for u in range(_UNROLL):
                    sl = pl.ds(b + u * _L, _L)
                    drow[sl] = srow[sl] * _SCALE
                return inner

            lax.fori_loop(0, _NVEC, vec, 0)
            return outer

        lax.fori_loop(0, _CROWS, row, 0)

    for c in range(_NBUF):
        in_copy(c, c).start()

    _NGRP = _NCHUNK // _NBUF

    def grp(g, carry):
        for p in range(_NBUF):
            c = g * _NBUF + p

            @pl.when(g >= 1)
            def _():
                out_copy(c - _NBUF, p).wait()

            in_copy(c, p).wait()
            compute(p)
            out_copy(c, p).start()

            @pl.when(g < _NGRP - 1)
            def _():
                in_copy(c + _NBUF, p).start()

        return carry

    lax.fori_loop(0, _NGRP, grp, 0)

    for c in range(_NCHUNK - _NBUF, _NCHUNK):
        out_copy(c, c % _NBUF).wait()


def kernel(x, emb):
    del x
    return _sc_scale_copy(emb)
